# baseline (speedup 1.0000x reference)
import functools

import jax
import jax.numpy as jnp
from jax import lax
from jax.experimental import pallas as pl
from jax.experimental.pallas import tpu as pltpu
from jax.experimental.pallas import tpu_sc as plsc

VOCAB = 100000
VE_DIM = 1024
MODEL_DIM = 4096

NUM_CORES = 2
NUM_SUBCORES = 16
NUM_WORKERS = NUM_CORES * NUM_SUBCORES

CHUNK = 32


def _gather_sc(table, idx_flat):
    n_tok = idx_flat.shape[0]
    per_w = n_tok // NUM_WORKERS
    n_chunks = per_w // CHUNK
    mesh = plsc.VectorSubcoreMesh(core_axis_name="c", subcore_axis_name="s")

    @functools.partial(
        pl.kernel,
        mesh=mesh,
        out_type=jax.ShapeDtypeStruct((n_tok, VE_DIM), jnp.float32),
        scratch_types=[
            pltpu.VMEM((per_w,), jnp.int32),
            pltpu.VMEM((CHUNK, VE_DIM), jnp.float32),
            pltpu.VMEM((CHUNK, VE_DIM), jnp.float32),
            pltpu.SemaphoreType.DMA,
            pltpu.SemaphoreType.DMA,
        ],
    )
    def gather_kernel(table_hbm, idx_hbm, out_hbm, idx_v, rows0, rows1, sem0, sem1):
        wid = lax.axis_index("s") * NUM_CORES + lax.axis_index("c")
        base = wid * per_w
        pltpu.sync_copy(idx_hbm.at[pl.ds(base, per_w)], idx_v)

        bufs = (rows0, rows1)
        sems = (sem0, sem1)
        copies = [None, None]
        for c in range(n_chunks):
            b = c % 2
            copies[b] = pltpu.async_copy(
                table_hbm.at[idx_v.at[pl.ds(c * CHUNK, CHUNK)]], bufs[b], sems[b]
            )
            if c > 0:
                pb = (c - 1) % 2
                copies[pb].wait()
                pltpu.sync_copy(
                    bufs[pb], out_hbm.at[pl.ds(base + (c - 1) * CHUNK, CHUNK)]
                )
        lb = (n_chunks - 1) % 2
        copies[lb].wait()
        pltpu.sync_copy(
            bufs[lb], out_hbm.at[pl.ds(base + (n_chunks - 1) * CHUNK, CHUNK)]
        )

    return gather_kernel(table, idx_flat)


BM = 512
BN = 1024


def _mm_kernel(h_ref, w_ref, scale_ref, o_ref):
    acc = lax.dot_general(
        h_ref[...].astype(jnp.bfloat16),
        w_ref[...].astype(jnp.bfloat16),
        (((1,), (1,)), ((), ())),
        preferred_element_type=jnp.float32,
    )
    o_ref[...] = acc * scale_ref[0]


def _project_tc(h, w, scale):
    n_tok = h.shape[0]
    grid = (MODEL_DIM // BN, n_tok // BM)
    return pl.pallas_call(
        _mm_kernel,
        grid=grid,
        in_specs=[
            pl.BlockSpec((BM, VE_DIM), lambda n, m: (m, 0)),
            pl.BlockSpec((BN, VE_DIM), lambda n, m: (n, 0)),
            pl.BlockSpec(memory_space=pltpu.SMEM),
        ],
        out_specs=pl.BlockSpec((BM, BN), lambda n, m: (m, n)),
        out_shape=jax.ShapeDtypeStruct((n_tok, MODEL_DIM), jnp.float32),
    )(h, w, scale.reshape((1,)))


def kernel(token_ids, embed_weight, proj_weight, scale):
    b, s = token_ids.shape
    idx_flat = token_ids.reshape((b * s,)).astype(jnp.int32)
    h = _gather_sc(embed_weight, idx_flat)
    out = _project_tc(h, proj_weight, scale.astype(jnp.float32))
    return out.reshape((b, s, MODEL_DIM))


# speedup vs baseline: 1.3700x; 1.3700x over previous
import functools

import jax
import jax.numpy as jnp
from jax import lax
from jax.experimental import pallas as pl
from jax.experimental.pallas import tpu as pltpu
from jax.experimental.pallas import tpu_sc as plsc

VOCAB = 100000
VE_DIM = 1024
MODEL_DIM = 4096

NUM_CORES = 2
NUM_SUBCORES = 16
NUM_WORKERS = NUM_CORES * NUM_SUBCORES

CHUNK = 32


def _gather_sc(table, idx_flat):
    n_tok = idx_flat.shape[0]
    per_w = n_tok // NUM_WORKERS
    n_chunks = per_w // CHUNK
    mesh = plsc.VectorSubcoreMesh(core_axis_name="c", subcore_axis_name="s")

    @functools.partial(
        pl.kernel,
        mesh=mesh,
        out_type=jax.ShapeDtypeStruct((n_tok, VE_DIM), jnp.float32),
        scratch_types=[
            pltpu.VMEM((per_w,), jnp.int32),
            pltpu.VMEM((CHUNK, VE_DIM), jnp.float32),
            pltpu.VMEM((CHUNK, VE_DIM), jnp.float32),
            pltpu.SemaphoreType.DMA,
            pltpu.SemaphoreType.DMA,
        ],
    )
    def gather_kernel(table_hbm, idx_hbm, out_hbm, idx_v, rows0, rows1, sem0, sem1):
        wid = lax.axis_index("s") * NUM_CORES + lax.axis_index("c")
        base = wid * per_w
        pltpu.sync_copy(idx_hbm.at[pl.ds(base, per_w)], idx_v)

        bufs = (rows0, rows1)
        sems = (sem0, sem1)
        copies = [None, None]
        for c in range(n_chunks):
            b = c % 2
            copies[b] = pltpu.async_copy(
                table_hbm.at[idx_v.at[pl.ds(c * CHUNK, CHUNK)]], bufs[b], sems[b]
            )
            if c > 0:
                pb = (c - 1) % 2
                copies[pb].wait()
                pltpu.sync_copy(
                    bufs[pb], out_hbm.at[pl.ds(base + (c - 1) * CHUNK, CHUNK)]
                )
        lb = (n_chunks - 1) % 2
        copies[lb].wait()
        pltpu.sync_copy(
            bufs[lb], out_hbm.at[pl.ds(base + (n_chunks - 1) * CHUNK, CHUNK)]
        )

    return gather_kernel(table, idx_flat)


BM = 512


def _mm_kernel(h_ref, w_ref, scale_ref, o_ref, w_bf):
    @pl.when(pl.program_id(0) == 0)
    def _():
        w_bf[...] = w_ref[...].astype(jnp.bfloat16)

    acc = lax.dot_general(
        h_ref[...].astype(jnp.bfloat16),
        w_bf[...],
        (((1,), (1,)), ((), ())),
        preferred_element_type=jnp.float32,
    )
    o_ref[...] = acc * scale_ref[0]


def _project_tc(h, w, scale):
    n_tok = h.shape[0]
    grid = (n_tok // BM,)
    return pl.pallas_call(
        _mm_kernel,
        grid=grid,
        in_specs=[
            pl.BlockSpec((BM, VE_DIM), lambda m: (m, 0)),
            pl.BlockSpec((MODEL_DIM, VE_DIM), lambda m: (0, 0)),
            pl.BlockSpec(memory_space=pltpu.SMEM),
        ],
        out_specs=pl.BlockSpec((BM, MODEL_DIM), lambda m: (m, 0)),
        out_shape=jax.ShapeDtypeStruct((n_tok, MODEL_DIM), jnp.float32),
        scratch_shapes=[pltpu.VMEM((MODEL_DIM, VE_DIM), jnp.bfloat16)],
    )(h, w, scale.reshape((1,)))


def kernel(token_ids, embed_weight, proj_weight, scale):
    b, s = token_ids.shape
    idx_flat = token_ids.reshape((b * s,)).astype(jnp.int32)
    h = _gather_sc(embed_weight, idx_flat)
    out = _project_tc(h, proj_weight, scale.astype(jnp.float32))
    return out.reshape((b, s, MODEL_DIM))
